# baseline (device time: 79350 ns/iter reference)
import jax
import jax.numpy as jnp
from jax import lax
from jax.experimental import pallas as pl
from jax.experimental.pallas import tpu as pltpu

B, S, D = 1, 1024, 2048
H, Dh, Dr = 16, 128, 32
DC_SHARD = 128
SCALE = (Dh + Dr) ** -0.5
NBLK = 4
BLK = D // NBLK

_VMEM = pl.BlockSpec(memory_space=pltpu.VMEM)
_BF = jnp.bfloat16


def _dot(a, b):
    return jnp.dot(a, b, preferred_element_type=jnp.float32)


def _dot_t(a, b):
    return lax.dot_general(a, b, (((1,), (1,)), ((), ())),
                           preferred_element_type=jnp.float32)


def _comm_proj_body(x_ref, wdkv_ref, wuk_ref, wuv_ref, wq_ref, wqr_ref,
                    wkr_ref, q_ref, qr_ref, kr_ref, k_ref, v_ref,
                    xb_s, c_mine, c_peer, wuk_b, wuv_b, wuk_peer, wuv_peer,
                    send_sems, recv_sems):
    j = pl.program_id(0)
    my_x = lax.axis_index("x")
    my_y = lax.axis_index("y")
    my_z = lax.axis_index("z")
    peer = (1 - my_x, my_y, my_z)

    def _rdmas():
        out = []
        for i, (src, dst) in enumerate(
            [(wuk_b, wuk_peer), (wuv_b, wuv_peer), (c_mine, c_peer)]
        ):
            out.append(pltpu.make_async_remote_copy(
                src_ref=src, dst_ref=dst,
                send_sem=send_sems.at[i], recv_sem=recv_sems.at[i],
                device_id=peer, device_id_type=pl.DeviceIdType.MESH,
            ))
        return out

    @pl.when(j == 0)
    def _():
        barrier_sem = pltpu.get_barrier_semaphore()
        pl.semaphore_signal(barrier_sem, inc=1, device_id=peer,
                            device_id_type=pl.DeviceIdType.MESH)
        pl.semaphore_wait(barrier_sem, 1)

        xb_s[...] = x_ref[0].astype(_BF)
        wuk_b[...] = wuk_ref[...].astype(_BF)
        wuv_b[...] = wuv_ref[...].astype(_BF)
        c_mine[...] = _dot(xb_s[...], wdkv_ref[...].astype(_BF)).astype(_BF)
        for r in _rdmas():
            r.start()

        qr_ref[...] = (
            _dot(xb_s[...], wqr_ref[...].astype(_BF)) * SCALE).astype(_BF)
        kr_ref[...] = _dot(xb_s[...], wkr_ref[...].astype(_BF)).astype(_BF)

    q_ref[...] = (_dot(xb_s[...], wq_ref[...].astype(_BF)) * SCALE).astype(_BF)

    @pl.when(j == NBLK - 1)
    def _():
        for r in _rdmas():
            r.wait()
        k_ref[...] = (_dot(c_mine[...], wuk_b[...])
                      + _dot(c_peer[...], wuk_peer[...])).astype(_BF)
        v_ref[...] = (_dot(c_mine[...], wuv_b[...])
                      + _dot(c_peer[...], wuv_peer[...])).astype(_BF)


DE = 256


def _attn_body(q_ref, qr_ref, kr_ref, k_ref, v_ref, o_ref,
               qext, kext, vext):
    qext[...] = jnp.zeros((S, H * DE), dtype=_BF)
    kext[...] = jnp.zeros((S, H * DE), dtype=_BF)
    vext[...] = jnp.ones((S, H * DE), dtype=_BF)
    kr = kr_ref[...]
    for h in range(H):
        qext[:, h * DE:h * DE + Dh] = q_ref[:, h * Dh:(h + 1) * Dh]
        qext[:, h * DE + Dh:h * DE + Dh + Dr] = qr_ref[:, h * Dr:(h + 1) * Dr]
        kext[:, h * DE:h * DE + Dh] = k_ref[:, h * Dh:(h + 1) * Dh]
        kext[:, h * DE + Dh:h * DE + Dh + Dr] = kr
        vext[:, h * DE:h * DE + Dh] = v_ref[:, h * Dh:(h + 1) * Dh]
    for h in range(H):
        p = jnp.exp(_dot_t(qext[:, h * DE:(h + 1) * DE],
                           kext[:, h * DE:(h + 1) * DE]).astype(_BF))
        oext = _dot(p, vext[:, h * DE:(h + 1) * DE])
        o_ref[:, h * Dh:(h + 1) * Dh] = (
            oext[:, :Dh] / oext[:, Dh:Dh + 1]).astype(_BF)


def _out_body(o_ref, wo_ref, out_ref):
    out_ref[0] = _dot(o_ref[...], wo_ref[...].astype(_BF))


def kernel(x, Wdkv, Wuk, Wuv, Wq, Wqr, Wkr, Wo):
    f32 = jnp.float32

    Q, Qr, Kr, K, V = pl.pallas_call(
        _comm_proj_body,
        grid=(NBLK,),
        out_shape=[jax.ShapeDtypeStruct((S, D), _BF),
                   jax.ShapeDtypeStruct((S, H * Dr), _BF),
                   jax.ShapeDtypeStruct((S, Dr), _BF),
                   jax.ShapeDtypeStruct((S, D), _BF),
                   jax.ShapeDtypeStruct((S, D), _BF)],
        in_specs=[
            _VMEM,
            _VMEM,
            _VMEM,
            _VMEM,
            pl.BlockSpec((D, BLK), lambda j: (0, j)),
            _VMEM,
            _VMEM,
        ],
        out_specs=[
            pl.BlockSpec((S, BLK), lambda j: (0, j)),
            pl.BlockSpec((S, H * Dr), lambda j: (0, 0)),
            pl.BlockSpec((S, Dr), lambda j: (0, 0)),
            pl.BlockSpec((S, D), lambda j: (0, 0)),
            pl.BlockSpec((S, D), lambda j: (0, 0)),
        ],
        scratch_shapes=[
            pltpu.VMEM((S, D), _BF),
            pltpu.VMEM((S, DC_SHARD), _BF),
            pltpu.VMEM((S, DC_SHARD), _BF),
            pltpu.VMEM((DC_SHARD, D), _BF),
            pltpu.VMEM((DC_SHARD, D), _BF),
            pltpu.VMEM((DC_SHARD, D), _BF),
            pltpu.VMEM((DC_SHARD, D), _BF),
            pltpu.SemaphoreType.DMA((3,)),
            pltpu.SemaphoreType.DMA((3,)),
        ],
        compiler_params=pltpu.CompilerParams(
            collective_id=0, has_side_effects=True),
    )(x, Wdkv, Wuk, Wuv, Wq, Wqr, Wkr)

    O = pl.pallas_call(
        _attn_body,
        out_shape=jax.ShapeDtypeStruct((S, D), _BF),
        in_specs=[_VMEM] * 5,
        out_specs=_VMEM,
        scratch_shapes=[
            pltpu.VMEM((S, H * DE), _BF),
            pltpu.VMEM((S, H * DE), _BF),
            pltpu.VMEM((S, H * DE), _BF),
        ],
    )(Q, Qr, Kr, K, V)

    out = pl.pallas_call(
        _out_body,
        grid=(NBLK,),
        out_shape=jax.ShapeDtypeStruct((B, S, D), f32),
        in_specs=[
            _VMEM,
            pl.BlockSpec((D, BLK), lambda j: (0, j)),
        ],
        out_specs=pl.BlockSpec((B, S, BLK), lambda j: (0, 0, j)),
    )(O, Wo)
    return out


# device time: 74430 ns/iter; 1.0661x vs baseline; 1.0661x over previous
import jax
import jax.numpy as jnp
from jax import lax
from jax.experimental import pallas as pl
from jax.experimental.pallas import tpu as pltpu

B, S, D = 1, 1024, 2048
H, Dh, Dr = 16, 128, 32
DC_SHARD = 128
SCALE = (Dh + Dr) ** -0.5
DE = 256
NBLK = 8
BLK = D // NBLK
HPB = BLK // Dh
NBLK_O = 4
BLK_O = D // NBLK_O

_VMEM = pl.BlockSpec(memory_space=pltpu.VMEM)
_BF = jnp.bfloat16


def _dot(a, b):
    return jnp.dot(a, b, preferred_element_type=jnp.float32)


def _dot_t(a, b):
    return lax.dot_general(a, b, (((1,), (1,)), ((), ())),
                           preferred_element_type=jnp.float32)


def _mla_body(xb_ref, wdkv_ref, wuk_ref, wuv_ref, wq_ref, wqr_ref,
              wkr_ref, o_ref,
              qext, kx_s, vx_s, kr_s, c_mine, c_peer, wuk_b, wuv_b,
              wuk_peer, wuv_peer, send_sems, recv_sems):
    j = pl.program_id(0)
    my_x = lax.axis_index("x")
    my_y = lax.axis_index("y")
    my_z = lax.axis_index("z")
    peer = (1 - my_x, my_y, my_z)
    xb = xb_ref[0]

    def _rdmas():
        out = []
        for i, (src, dst) in enumerate(
            [(wuk_b, wuk_peer), (wuv_b, wuv_peer), (c_mine, c_peer)]
        ):
            out.append(pltpu.make_async_remote_copy(
                src_ref=src, dst_ref=dst,
                send_sem=send_sems.at[i], recv_sem=recv_sems.at[i],
                device_id=peer, device_id_type=pl.DeviceIdType.MESH,
            ))
        return out

    @pl.when(j == 0)
    def _():
        barrier_sem = pltpu.get_barrier_semaphore()
        pl.semaphore_signal(barrier_sem, inc=1, device_id=peer,
                            device_id_type=pl.DeviceIdType.MESH)
        pl.semaphore_wait(barrier_sem, 1)

        wuk_b[...] = wuk_ref[...].astype(_BF)
        wuv_b[...] = wuv_ref[...].astype(_BF)
        c_mine[...] = _dot(xb, wdkv_ref[...].astype(_BF)).astype(_BF)
        for r in _rdmas():
            r.start()

        qext[...] = jnp.zeros((S, H * DE), dtype=_BF)
        qr = (_dot(xb, wqr_ref[...].astype(_BF)) * SCALE).astype(_BF)
        for h in range(H):
            qext[:, h * DE + Dh:h * DE + Dh + Dr] = qr[:, h * Dr:(h + 1) * Dr]
        kr_s[...] = _dot(xb, wkr_ref[...].astype(_BF)).astype(_BF)

    qblk = (_dot(xb, wq_ref[...].astype(_BF)) * SCALE).astype(_BF)
    for i in range(HPB):
        h = HPB * j + i
        qext[:, pl.ds(h * DE, Dh)] = qblk[:, i * Dh:(i + 1) * Dh]

    @pl.when(j == NBLK - 1)
    def _():
        for r in _rdmas():
            r.wait()
        for b in range(2):
            kx_s[b, :, Dh:Dh + Dr] = kr_s[...]
            kx_s[b, :, Dh + Dr:] = jnp.zeros((S, DE - Dh - Dr), dtype=_BF)
            vx_s[b, :, Dh:] = jnp.ones((S, DE - Dh), dtype=_BF)
        for g in range(H // 2):
            cols = slice(g * 2 * Dh, (g + 1) * 2 * Dh)
            kc = (_dot(c_mine[...], wuk_b[:, cols])
                  + _dot(c_peer[...], wuk_peer[:, cols])).astype(_BF)
            vc = (_dot(c_mine[...], wuv_b[:, cols])
                  + _dot(c_peer[...], wuv_peer[:, cols])).astype(_BF)
            for i in range(2):
                h = 2 * g + i
                b = h % 2
                kx_s[b, :, :Dh] = kc[:, i * Dh:(i + 1) * Dh]
                vx_s[b, :, :Dh] = vc[:, i * Dh:(i + 1) * Dh]
                p = jnp.exp(_dot_t(qext[:, h * DE:(h + 1) * DE],
                                   kx_s[b]).astype(_BF))
                oext = _dot(p, vx_s[b])
                o_ref[:, h * Dh:(h + 1) * Dh] = (
                    oext[:, :Dh] / oext[:, Dh:Dh + 1]).astype(_BF)


def _out_body(o_ref, wo_ref, out_ref):
    out_ref[0] = _dot(o_ref[...], wo_ref[...].astype(_BF))


def kernel(x, Wdkv, Wuk, Wuv, Wq, Wqr, Wkr, Wo):
    f32 = jnp.float32
    xb = x.astype(_BF)

    O = pl.pallas_call(
        _mla_body,
        grid=(NBLK,),
        out_shape=jax.ShapeDtypeStruct((S, D), _BF),
        in_specs=[
            _VMEM,
            _VMEM,
            _VMEM,
            _VMEM,
            pl.BlockSpec((D, BLK), lambda j: (0, j)),
            _VMEM,
            _VMEM,
        ],
        out_specs=pl.BlockSpec((S, D), lambda j: (0, 0)),
        scratch_shapes=[
            pltpu.VMEM((S, H * DE), _BF),
            pltpu.VMEM((2, S, DE), _BF),
            pltpu.VMEM((2, S, DE), _BF),
            pltpu.VMEM((S, Dr), _BF),
            pltpu.VMEM((S, DC_SHARD), _BF),
            pltpu.VMEM((S, DC_SHARD), _BF),
            pltpu.VMEM((DC_SHARD, D), _BF),
            pltpu.VMEM((DC_SHARD, D), _BF),
            pltpu.VMEM((DC_SHARD, D), _BF),
            pltpu.VMEM((DC_SHARD, D), _BF),
            pltpu.SemaphoreType.DMA((3,)),
            pltpu.SemaphoreType.DMA((3,)),
        ],
        compiler_params=pltpu.CompilerParams(
            collective_id=0, has_side_effects=True),
    )(xb, Wdkv, Wuk, Wuv, Wq, Wqr, Wkr)

    out = pl.pallas_call(
        _out_body,
        grid=(NBLK_O,),
        out_shape=jax.ShapeDtypeStruct((B, S, D), f32),
        in_specs=[
            _VMEM,
            pl.BlockSpec((D, BLK_O), lambda j: (0, j)),
        ],
        out_specs=pl.BlockSpec((B, S, BLK_O), lambda j: (0, 0, j)),
    )(O, Wo)
    return out
